# baseline (speedup 1.0000x reference)
import functools
import math

import jax
import jax.numpy as jnp
from jax.experimental import pallas as pl

B, S, DIM, HEADS = 1, 2048, 1024, 16
HEAD = DIM // HEADS
TEN, TOPK, EDIM, SDIM = 8, 2, 512, 1024
EPS, THETA, RSF = 1e-5, 10000.0, 1.0

SB = 256
QB = 512
NEG = -1e30

f32 = jnp.float32
bf16 = jnp.bfloat16


def _k1_body(x_ref, w_ref, qkvw_ref, out_ref):
    x = x_ref[...]
    xn = x * jax.lax.rsqrt(jnp.mean(x * x, axis=-1, keepdims=True) + EPS)
    xn = xn * w_ref[...]
    out_ref[...] = jnp.dot(xn.astype(bf16), qkvw_ref[...],
                           preferred_element_type=f32).astype(bf16)


def _k1(x, attn_norm_w, qkv_wb):
    return pl.pallas_call(
        _k1_body,
        grid=(S // SB,),
        in_specs=[
            pl.BlockSpec((SB, DIM), lambda i: (i, 0)),
            pl.BlockSpec((1, DIM), lambda i: (0, 0)),
            pl.BlockSpec((DIM, 3 * DIM), lambda i: (0, 0)),
        ],
        out_specs=pl.BlockSpec((SB, 3 * DIM), lambda i: (i, 0)),
        out_shape=jax.ShapeDtypeStruct((S, 3 * DIM), bf16),
    )(x, attn_norm_w, qkv_wb)


def _prep_qk(x, pos0):
    n = jnp.sqrt(jnp.sum(x * x, axis=-1, keepdims=True))
    x = x / jnp.maximum(n, EPS)
    rows = x.shape[0]
    half = HEAD // 2
    pos = jax.lax.broadcasted_iota(jnp.int32, (rows, half), 0).astype(f32) + pos0
    k = jax.lax.broadcasted_iota(jnp.int32, (rows, half), 1).astype(f32)
    inv = jnp.exp(k * (2.0 / HEAD) * math.log(1.0 / THETA))
    fr = pos * inv
    c, s = jnp.cos(fr), jnp.sin(fr)
    x1, x2 = x[:, :half], x[:, half:]
    return jnp.concatenate([x1 * c + x2 * s, -x1 * s + x2 * c], axis=-1)


def _attn_one_head(q, k, v, qi):
    q = _prep_qk(q.astype(f32), qi * QB).astype(bf16)
    k = _prep_qk(k.astype(f32), 0).astype(bf16)
    logits = jax.lax.dot_general(q, k, (((1,), (1,)), ((), ())),
                                 preferred_element_type=f32)
    logits = logits * (1.0 / math.sqrt(float(HEAD)))
    row = jax.lax.broadcasted_iota(jnp.int32, (QB, S), 0) + qi * QB
    col = jax.lax.broadcasted_iota(jnp.int32, (QB, S), 1)
    logits = jnp.where(col <= row, logits, NEG)
    m = jnp.max(logits, axis=-1, keepdims=True)
    p = jnp.exp(logits - m)
    p = p / jnp.sum(p, axis=-1, keepdims=True)
    return jnp.dot(p.astype(bf16), v, preferred_element_type=f32)


def _k2_body(q_ref, k_ref, v_ref, out_ref):
    qi = pl.program_id(1)
    outs = []
    for i in range(2):
        sl = slice(i * HEAD, (i + 1) * HEAD)
        outs.append(_attn_one_head(q_ref[:, sl], k_ref[:, sl], v_ref[:, sl], qi))
    out_ref[...] = jnp.concatenate(outs, axis=-1).astype(bf16)


def _k2(qkv):
    HP = HEADS // 2
    return pl.pallas_call(
        _k2_body,
        grid=(HP, S // QB),
        in_specs=[
            pl.BlockSpec((QB, 2 * HEAD), lambda h, qi: (qi, h)),
            pl.BlockSpec((S, 2 * HEAD), lambda h, qi: (0, HP + h)),
            pl.BlockSpec((S, 2 * HEAD), lambda h, qi: (0, 2 * HP + h)),
        ],
        out_specs=pl.BlockSpec((QB, 2 * HEAD), lambda h, qi: (qi, h)),
        out_shape=jax.ShapeDtypeStruct((S, DIM), bf16),
    )(qkv, qkv, qkv)


def _k3_body(attn_ref, ow_ref, x_ref, fw_ref, keys_ref, idx_ref, val_ref,
             resid_ref, xffn_ref, scores_ref):
    att = jnp.dot(attn_ref[...], ow_ref[...], preferred_element_type=f32)
    resid = att + x_ref[...]
    resid_ref[...] = resid
    xn = resid * jax.lax.rsqrt(jnp.mean(resid * resid, axis=-1, keepdims=True) + EPS)
    xn = xn * fw_ref[...]
    xffn_ref[...] = xn.astype(bf16)
    tv = jnp.dot(xn, keys_ref[...], preferred_element_type=f32)
    idx = idx_ref[...]
    tvsel = jnp.zeros_like(tv)
    for e in range(TEN):
        tvsel = tvsel + tv[:, e:e + 1] * (idx == e).astype(f32)
    scores_ref[...] = jax.nn.sigmoid(val_ref[...] + tvsel) * RSF


def _k3(attn, o_wb, x_input, ffn_norm_w, keys_pad, idx_pad, val_pad):
    return pl.pallas_call(
        _k3_body,
        grid=(S // SB,),
        in_specs=[
            pl.BlockSpec((SB, DIM), lambda i: (i, 0)),
            pl.BlockSpec((DIM, DIM), lambda i: (0, 0)),
            pl.BlockSpec((SB, DIM), lambda i: (i, 0)),
            pl.BlockSpec((1, DIM), lambda i: (0, 0)),
            pl.BlockSpec((DIM, 128), lambda i: (0, 0)),
            pl.BlockSpec((SB, 128), lambda i: (i, 0)),
            pl.BlockSpec((SB, 128), lambda i: (i, 0)),
        ],
        out_specs=[
            pl.BlockSpec((SB, DIM), lambda i: (i, 0)),
            pl.BlockSpec((SB, DIM), lambda i: (i, 0)),
            pl.BlockSpec((SB, 128), lambda i: (i, 0)),
        ],
        out_shape=[
            jax.ShapeDtypeStruct((S, DIM), f32),
            jax.ShapeDtypeStruct((S, DIM), bf16),
            jax.ShapeDtypeStruct((S, 128), f32),
        ],
    )(attn, o_wb, x_input, ffn_norm_w, keys_pad, idx_pad, val_pad)


def _k4_body(x_ref, w0_ref, w1_ref, w2t_ref, idx_ref, sc_ref, y_ref):
    e = pl.program_id(0)
    x = x_ref[...]
    g = jnp.dot(x, w0_ref[0], preferred_element_type=f32)
    u = jnp.dot(x, w1_ref[0], preferred_element_type=f32)
    h = (jax.nn.silu(g) * u).astype(bf16)
    o = jnp.dot(h, w2t_ref[0], preferred_element_type=f32)
    w = jnp.sum(sc_ref[...] * (idx_ref[...] == e).astype(f32), axis=-1,
                keepdims=True)
    contrib = o * w

    @pl.when(e == 0)
    def _():
        y_ref[...] = contrib

    @pl.when(e > 0)
    def _():
        y_ref[...] = y_ref[...] + contrib


def _k4(xffn, w0b, w1b, w2tb, idx_pad, scores):
    return pl.pallas_call(
        _k4_body,
        grid=(TEN,),
        in_specs=[
            pl.BlockSpec((S, DIM), lambda e: (0, 0)),
            pl.BlockSpec((1, DIM, EDIM), lambda e: (e, 0, 0)),
            pl.BlockSpec((1, DIM, EDIM), lambda e: (e, 0, 0)),
            pl.BlockSpec((1, EDIM, DIM), lambda e: (e, 0, 0)),
            pl.BlockSpec((S, 128), lambda e: (0, 0)),
            pl.BlockSpec((S, 128), lambda e: (0, 0)),
        ],
        out_specs=pl.BlockSpec((S, DIM), lambda e: (0, 0)),
        out_shape=jax.ShapeDtypeStruct((S, DIM), f32),
    )(xffn, w0b, w1b, w2tb, idx_pad, scores)


def _k5_body(x_ref, up_ref, down_ref, y_ref, resid_ref, out_ref):
    xu = jnp.dot(x_ref[...], up_ref[...], preferred_element_type=f32)
    x1, x2 = xu[:, :SDIM], xu[:, SDIM:]
    h = (jax.nn.silu(x1) * x2).astype(bf16)
    ys = jnp.dot(h, down_ref[...], preferred_element_type=f32)
    out_ref[...] = ys + y_ref[...] + resid_ref[...]


def _k5(xffn, up_wb, down_wb, y_moe, resid):
    return pl.pallas_call(
        _k5_body,
        grid=(S // SB,),
        in_specs=[
            pl.BlockSpec((SB, DIM), lambda i: (i, 0)),
            pl.BlockSpec((DIM, 2 * SDIM), lambda i: (0, 0)),
            pl.BlockSpec((SDIM, DIM), lambda i: (0, 0)),
            pl.BlockSpec((SB, DIM), lambda i: (i, 0)),
            pl.BlockSpec((SB, DIM), lambda i: (i, 0)),
        ],
        out_specs=pl.BlockSpec((SB, DIM), lambda i: (i, 0)),
        out_shape=jax.ShapeDtypeStruct((S, DIM), f32),
    )(xffn, up_wb, down_wb, y_moe, resid)


def kernel(x_input, indices, values, attn_norm_w, qkv_w, o_w, ffn_norm_w,
           keys_w, experts_w, up_w, down_w):
    x = x_input.reshape(S, DIM)
    qkv_wb = qkv_w.astype(bf16)
    o_wb = o_w.astype(bf16)
    w0b = experts_w[0].astype(bf16)
    w1b = experts_w[1].astype(bf16)
    w2tb = experts_w[2].transpose(0, 2, 1).astype(bf16)
    up_wb = up_w.astype(bf16)
    down_wb = down_w.astype(bf16)
    keys_pad = jnp.pad(keys_w, ((0, 0), (0, 128 - TEN)))
    idx_pad = jnp.pad(indices.astype(jnp.int32), ((0, 0), (0, 128 - TOPK)),
                      constant_values=TEN)
    val_pad = jnp.pad(values, ((0, 0), (0, 128 - TOPK)), constant_values=NEG)

    qkv = _k1(x, attn_norm_w.reshape(1, DIM), qkv_wb)
    attn = _k2(qkv)
    resid, xffn, scores = _k3(attn, o_wb, x, ffn_norm_w.reshape(1, DIM),
                              keys_pad, idx_pad, val_pad)
    y_moe = _k4(xffn, w0b, w1b, w2tb, idx_pad, scores)
    out = _k5(xffn, up_wb, down_wb, y_moe, resid)
    return out.reshape(B, S, DIM)


# speedup vs baseline: 1.2668x; 1.2668x over previous
import functools
import math

import jax
import jax.numpy as jnp
from jax.experimental import pallas as pl

B, S, DIM, HEADS = 1, 2048, 1024, 16
HEAD = DIM // HEADS
TEN, TOPK, EDIM, SDIM = 8, 2, 512, 1024
EPS, THETA, RSF = 1e-5, 10000.0, 1.0

SB = 256
QB = 512
NEG = -1e30

f32 = jnp.float32
bf16 = jnp.bfloat16


def _k1_body(x_ref, w_ref, qkvw_ref, out_ref):
    x = x_ref[...]
    xn = x * jax.lax.rsqrt(jnp.mean(x * x, axis=-1, keepdims=True) + EPS)
    xn = xn * w_ref[...]
    out_ref[...] = jnp.dot(xn.astype(bf16), qkvw_ref[...],
                           preferred_element_type=f32).astype(bf16)


def _k1(x, attn_norm_w, qkv_wb):
    return pl.pallas_call(
        _k1_body,
        grid=(S // SB,),
        in_specs=[
            pl.BlockSpec((SB, DIM), lambda i: (i, 0)),
            pl.BlockSpec((1, DIM), lambda i: (0, 0)),
            pl.BlockSpec((DIM, 3 * DIM), lambda i: (0, 0)),
        ],
        out_specs=pl.BlockSpec((SB, 3 * DIM), lambda i: (i, 0)),
        out_shape=jax.ShapeDtypeStruct((S, 3 * DIM), bf16),
    )(x, attn_norm_w, qkv_wb)


def _prep_qk(x, pos0):
    n = jnp.sqrt(jnp.sum(x * x, axis=-1, keepdims=True))
    x = x / jnp.maximum(n, EPS)
    rows = x.shape[0]
    half = HEAD // 2
    pos = jax.lax.broadcasted_iota(jnp.int32, (rows, half), 0).astype(f32) + pos0
    k = jax.lax.broadcasted_iota(jnp.int32, (rows, half), 1).astype(f32)
    inv = jnp.exp(k * (2.0 / HEAD) * math.log(1.0 / THETA))
    fr = pos * inv
    c, s = jnp.cos(fr), jnp.sin(fr)
    x1, x2 = x[:, :half], x[:, half:]
    return jnp.concatenate([x1 * c + x2 * s, -x1 * s + x2 * c], axis=-1)


def _attn_one_head(q_raw, k_ref, v_ref, sl, qi):
    scale = 1.0 / math.sqrt(float(HEAD))
    q = (_prep_qk(q_raw.astype(f32), qi * QB) * scale).astype(bf16)

    def chunk(kj, masked):
        kc = _prep_qk(k_ref[pl.ds(kj * QB, QB), sl].astype(f32),
                      kj * QB).astype(bf16)
        vc = v_ref[pl.ds(kj * QB, QB), sl]
        l = jax.lax.dot_general(q, kc, (((1,), (1,)), ((), ())),
                                preferred_element_type=f32)
        p = jnp.exp(l)
        if masked:
            row = jax.lax.broadcasted_iota(jnp.int32, (QB, QB), 0)
            col = jax.lax.broadcasted_iota(jnp.int32, (QB, QB), 1)
            p = jnp.where(col <= row, p, 0.0)
        o = jnp.dot(p.astype(bf16), vc, preferred_element_type=f32)
        return o, jnp.sum(p, axis=-1, keepdims=True)

    def body(kj, carry):
        o_acc, s_acc = carry
        o, s = chunk(kj, masked=False)
        return o_acc + o, s_acc + s

    o_acc, s_acc = jax.lax.fori_loop(
        0, qi, body,
        (jnp.zeros((QB, HEAD), f32), jnp.zeros((QB, 1), f32)))
    o, s = chunk(qi, masked=True)
    return (o_acc + o) / (s_acc + s)


def _k2_body(q_ref, k_ref, v_ref, out_ref):
    qi = pl.program_id(1)
    outs = []
    for i in range(2):
        sl = slice(i * HEAD, (i + 1) * HEAD)
        outs.append(_attn_one_head(q_ref[:, sl], k_ref, v_ref, sl, qi))
    out_ref[...] = jnp.concatenate(outs, axis=-1).astype(bf16)


def _k2(qkv):
    HP = HEADS // 2
    return pl.pallas_call(
        _k2_body,
        grid=(HP, S // QB),
        in_specs=[
            pl.BlockSpec((QB, 2 * HEAD), lambda h, qi: (qi, h)),
            pl.BlockSpec((S, 2 * HEAD), lambda h, qi: (0, HP + h)),
            pl.BlockSpec((S, 2 * HEAD), lambda h, qi: (0, 2 * HP + h)),
        ],
        out_specs=pl.BlockSpec((QB, 2 * HEAD), lambda h, qi: (qi, h)),
        out_shape=jax.ShapeDtypeStruct((S, DIM), bf16),
    )(qkv, qkv, qkv)


def _k3_body(attn_ref, ow_ref, x_ref, fw_ref, keys_ref, idx_ref, val_ref,
             resid_ref, xffn_ref, scores_ref):
    att = jnp.dot(attn_ref[...], ow_ref[...], preferred_element_type=f32)
    resid = att + x_ref[...]
    resid_ref[...] = resid
    xn = resid * jax.lax.rsqrt(jnp.mean(resid * resid, axis=-1, keepdims=True) + EPS)
    xn = xn * fw_ref[...]
    xffn_ref[...] = xn.astype(bf16)
    tv = jnp.dot(xn, keys_ref[...], preferred_element_type=f32)
    idx = idx_ref[...]
    tvsel = jnp.zeros_like(tv)
    for e in range(TEN):
        tvsel = tvsel + tv[:, e:e + 1] * (idx == e).astype(f32)
    scores_ref[...] = jax.nn.sigmoid(val_ref[...] + tvsel) * RSF


def _k3(attn, o_wb, x_input, ffn_norm_w, keys_pad, idx_pad, val_pad):
    return pl.pallas_call(
        _k3_body,
        grid=(S // SB,),
        in_specs=[
            pl.BlockSpec((SB, DIM), lambda i: (i, 0)),
            pl.BlockSpec((DIM, DIM), lambda i: (0, 0)),
            pl.BlockSpec((SB, DIM), lambda i: (i, 0)),
            pl.BlockSpec((1, DIM), lambda i: (0, 0)),
            pl.BlockSpec((DIM, 128), lambda i: (0, 0)),
            pl.BlockSpec((SB, 128), lambda i: (i, 0)),
            pl.BlockSpec((SB, 128), lambda i: (i, 0)),
        ],
        out_specs=[
            pl.BlockSpec((SB, DIM), lambda i: (i, 0)),
            pl.BlockSpec((SB, DIM), lambda i: (i, 0)),
            pl.BlockSpec((SB, 128), lambda i: (i, 0)),
        ],
        out_shape=[
            jax.ShapeDtypeStruct((S, DIM), f32),
            jax.ShapeDtypeStruct((S, DIM), bf16),
            jax.ShapeDtypeStruct((S, 128), f32),
        ],
    )(attn, o_wb, x_input, ffn_norm_w, keys_pad, idx_pad, val_pad)


def _k4_body(x_ref, w0_ref, w1_ref, w2t_ref, idx_ref, sc_ref, y_ref):
    e = pl.program_id(0)
    x = x_ref[...]
    g = jnp.dot(x, w0_ref[0], preferred_element_type=f32)
    u = jnp.dot(x, w1_ref[0], preferred_element_type=f32)
    h = (jax.nn.silu(g) * u).astype(bf16)
    o = jnp.dot(h, w2t_ref[0], preferred_element_type=f32)
    w = jnp.sum(sc_ref[...] * (idx_ref[...] == e).astype(f32), axis=-1,
                keepdims=True)
    contrib = o * w

    @pl.when(e == 0)
    def _():
        y_ref[...] = contrib

    @pl.when(e > 0)
    def _():
        y_ref[...] = y_ref[...] + contrib


def _k4(xffn, w0b, w1b, w2tb, idx_pad, scores):
    return pl.pallas_call(
        _k4_body,
        grid=(TEN,),
        in_specs=[
            pl.BlockSpec((S, DIM), lambda e: (0, 0)),
            pl.BlockSpec((1, DIM, EDIM), lambda e: (e, 0, 0)),
            pl.BlockSpec((1, DIM, EDIM), lambda e: (e, 0, 0)),
            pl.BlockSpec((1, EDIM, DIM), lambda e: (e, 0, 0)),
            pl.BlockSpec((S, 128), lambda e: (0, 0)),
            pl.BlockSpec((S, 128), lambda e: (0, 0)),
        ],
        out_specs=pl.BlockSpec((S, DIM), lambda e: (0, 0)),
        out_shape=jax.ShapeDtypeStruct((S, DIM), f32),
    )(xffn, w0b, w1b, w2tb, idx_pad, scores)


def _k5_body(x_ref, up_ref, down_ref, y_ref, resid_ref, out_ref):
    xu = jnp.dot(x_ref[...], up_ref[...], preferred_element_type=f32)
    x1, x2 = xu[:, :SDIM], xu[:, SDIM:]
    h = (jax.nn.silu(x1) * x2).astype(bf16)
    ys = jnp.dot(h, down_ref[...], preferred_element_type=f32)
    out_ref[...] = ys + y_ref[...] + resid_ref[...]


def _k5(xffn, up_wb, down_wb, y_moe, resid):
    return pl.pallas_call(
        _k5_body,
        grid=(S // SB,),
        in_specs=[
            pl.BlockSpec((SB, DIM), lambda i: (i, 0)),
            pl.BlockSpec((DIM, 2 * SDIM), lambda i: (0, 0)),
            pl.BlockSpec((SDIM, DIM), lambda i: (0, 0)),
            pl.BlockSpec((SB, DIM), lambda i: (i, 0)),
            pl.BlockSpec((SB, DIM), lambda i: (i, 0)),
        ],
        out_specs=pl.BlockSpec((SB, DIM), lambda i: (i, 0)),
        out_shape=jax.ShapeDtypeStruct((S, DIM), f32),
    )(xffn, up_wb, down_wb, y_moe, resid)


def kernel(x_input, indices, values, attn_norm_w, qkv_w, o_w, ffn_norm_w,
           keys_w, experts_w, up_w, down_w):
    x = x_input.reshape(S, DIM)
    qkv_wb = qkv_w.astype(bf16)
    o_wb = o_w.astype(bf16)
    w0b = experts_w[0].astype(bf16)
    w1b = experts_w[1].astype(bf16)
    w2tb = experts_w[2].transpose(0, 2, 1).astype(bf16)
    up_wb = up_w.astype(bf16)
    down_wb = down_w.astype(bf16)
    keys_pad = jnp.pad(keys_w, ((0, 0), (0, 128 - TEN)))
    idx_pad = jnp.pad(indices.astype(jnp.int32), ((0, 0), (0, 128 - TOPK)),
                      constant_values=TEN)
    val_pad = jnp.pad(values, ((0, 0), (0, 128 - TOPK)), constant_values=NEG)

    qkv = _k1(x, attn_norm_w.reshape(1, DIM), qkv_wb)
    attn = _k2(qkv)
    resid, xffn, scores = _k3(attn, o_wb, x, ffn_norm_w.reshape(1, DIM),
                              keys_pad, idx_pad, val_pad)
    y_moe = _k4(xffn, w0b, w1b, w2tb, idx_pad, scores)
    out = _k5(xffn, up_wb, down_wb, y_moe, resid)
    return out.reshape(B, S, DIM)


# speedup vs baseline: 1.3189x; 1.0411x over previous
import functools
import math

import jax
import jax.numpy as jnp
from jax.experimental import pallas as pl

B, S, DIM, HEADS = 1, 2048, 1024, 16
HEAD = DIM // HEADS
TEN, TOPK, EDIM, SDIM = 8, 2, 512, 1024
EPS, THETA, RSF = 1e-5, 10000.0, 1.0

SB = 256
QB = 512
NEG = -1e30

f32 = jnp.float32
bf16 = jnp.bfloat16


def _k1_body(x_ref, w_ref, qkvw_ref, out_ref):
    x = x_ref[...]
    xn = x * jax.lax.rsqrt(jnp.mean(x * x, axis=-1, keepdims=True) + EPS)
    xn = xn * w_ref[...]
    out_ref[...] = jnp.dot(xn.astype(bf16), qkvw_ref[...],
                           preferred_element_type=f32).astype(bf16)


def _k1(x, attn_norm_w, qkv_wb):
    return pl.pallas_call(
        _k1_body,
        grid=(S // SB,),
        in_specs=[
            pl.BlockSpec((SB, DIM), lambda i: (i, 0)),
            pl.BlockSpec((1, DIM), lambda i: (0, 0)),
            pl.BlockSpec((DIM, 3 * DIM), lambda i: (0, 0)),
        ],
        out_specs=pl.BlockSpec((SB, 3 * DIM), lambda i: (i, 0)),
        out_shape=jax.ShapeDtypeStruct((S, 3 * DIM), bf16),
    )(x, attn_norm_w, qkv_wb)


def _prep_qk(x, pos0):
    n = jnp.sqrt(jnp.sum(x * x, axis=-1, keepdims=True))
    x = x / jnp.maximum(n, EPS)
    rows = x.shape[0]
    half = HEAD // 2
    pos = jax.lax.broadcasted_iota(jnp.int32, (rows, half), 0).astype(f32) + pos0
    k = jax.lax.broadcasted_iota(jnp.int32, (rows, half), 1).astype(f32)
    inv = jnp.exp(k * (2.0 / HEAD) * math.log(1.0 / THETA))
    fr = pos * inv
    c, s = jnp.cos(fr), jnp.sin(fr)
    x1, x2 = x[:, :half], x[:, half:]
    return jnp.concatenate([x1 * c + x2 * s, -x1 * s + x2 * c], axis=-1)


def _attn_one_head(q_raw, k_ref, v_ref, sl, qi):
    scale = 1.0 / math.sqrt(float(HEAD))
    q = (_prep_qk(q_raw.astype(f32), qi * QB) * scale).astype(bf16)

    def chunk(kj, masked):
        kc = _prep_qk(k_ref[pl.ds(kj * QB, QB), sl].astype(f32),
                      kj * QB).astype(bf16)
        vc = v_ref[pl.ds(kj * QB, QB), sl]
        l = jax.lax.dot_general(q, kc, (((1,), (1,)), ((), ())),
                                preferred_element_type=f32)
        p = 1.0 + l * (1.0 + l * (0.5 + l * (1.0 / 6.0 + l * (1.0 / 24.0))))
        if masked:
            row = jax.lax.broadcasted_iota(jnp.int32, (QB, QB), 0)
            col = jax.lax.broadcasted_iota(jnp.int32, (QB, QB), 1)
            p = jnp.where(col <= row, p, 0.0)
        o = jnp.dot(p.astype(bf16), vc, preferred_element_type=f32)
        return o, jnp.sum(p, axis=-1, keepdims=True)

    def body(kj, carry):
        o_acc, s_acc = carry
        o, s = chunk(kj, masked=False)
        return o_acc + o, s_acc + s

    o_acc, s_acc = jax.lax.fori_loop(
        0, qi, body,
        (jnp.zeros((QB, HEAD), f32), jnp.zeros((QB, 1), f32)))
    o, s = chunk(qi, masked=True)
    return (o_acc + o) / (s_acc + s)


def _k2_body(q_ref, k_ref, v_ref, out_ref):
    qi = pl.program_id(1)
    outs = []
    for i in range(2):
        sl = slice(i * HEAD, (i + 1) * HEAD)
        outs.append(_attn_one_head(q_ref[:, sl], k_ref, v_ref, sl, qi))
    out_ref[...] = jnp.concatenate(outs, axis=-1).astype(bf16)


def _k2(qkv):
    HP = HEADS // 2
    return pl.pallas_call(
        _k2_body,
        grid=(HP, S // QB),
        in_specs=[
            pl.BlockSpec((QB, 2 * HEAD), lambda h, qi: (qi, h)),
            pl.BlockSpec((S, 2 * HEAD), lambda h, qi: (0, HP + h)),
            pl.BlockSpec((S, 2 * HEAD), lambda h, qi: (0, 2 * HP + h)),
        ],
        out_specs=pl.BlockSpec((QB, 2 * HEAD), lambda h, qi: (qi, h)),
        out_shape=jax.ShapeDtypeStruct((S, DIM), bf16),
    )(qkv, qkv, qkv)


def _k3_body(attn_ref, ow_ref, x_ref, fw_ref, keys_ref, idx_ref, val_ref,
             resid_ref, xffn_ref, scores_ref):
    att = jnp.dot(attn_ref[...], ow_ref[...], preferred_element_type=f32)
    resid = att + x_ref[...]
    resid_ref[...] = resid
    xn = resid * jax.lax.rsqrt(jnp.mean(resid * resid, axis=-1, keepdims=True) + EPS)
    xn = xn * fw_ref[...]
    xffn_ref[...] = xn.astype(bf16)
    tv = jnp.dot(xn, keys_ref[...], preferred_element_type=f32)
    idx = idx_ref[...]
    tvsel = jnp.zeros_like(tv)
    for e in range(TEN):
        tvsel = tvsel + tv[:, e:e + 1] * (idx == e).astype(f32)
    scores_ref[...] = jax.nn.sigmoid(val_ref[...] + tvsel) * RSF


def _k3(attn, o_wb, x_input, ffn_norm_w, keys_pad, idx_pad, val_pad):
    return pl.pallas_call(
        _k3_body,
        grid=(S // SB,),
        in_specs=[
            pl.BlockSpec((SB, DIM), lambda i: (i, 0)),
            pl.BlockSpec((DIM, DIM), lambda i: (0, 0)),
            pl.BlockSpec((SB, DIM), lambda i: (i, 0)),
            pl.BlockSpec((1, DIM), lambda i: (0, 0)),
            pl.BlockSpec((DIM, 128), lambda i: (0, 0)),
            pl.BlockSpec((SB, 128), lambda i: (i, 0)),
            pl.BlockSpec((SB, 128), lambda i: (i, 0)),
        ],
        out_specs=[
            pl.BlockSpec((SB, DIM), lambda i: (i, 0)),
            pl.BlockSpec((SB, DIM), lambda i: (i, 0)),
            pl.BlockSpec((SB, 128), lambda i: (i, 0)),
        ],
        out_shape=[
            jax.ShapeDtypeStruct((S, DIM), f32),
            jax.ShapeDtypeStruct((S, DIM), bf16),
            jax.ShapeDtypeStruct((S, 128), f32),
        ],
    )(attn, o_wb, x_input, ffn_norm_w, keys_pad, idx_pad, val_pad)


def _k4_body(x_ref, w0_ref, w1_ref, w2_ref, idx_ref, sc_ref, y_ref):
    e = pl.program_id(0)
    x = x_ref[...]
    g = jnp.dot(x, w0_ref[0].astype(bf16), preferred_element_type=f32)
    u = jnp.dot(x, w1_ref[0].astype(bf16), preferred_element_type=f32)
    h = (jax.nn.silu(g) * u).astype(bf16)
    o = jax.lax.dot_general(h, w2_ref[0].astype(bf16), (((1,), (1,)), ((), ())),
                            preferred_element_type=f32)
    w = jnp.sum(sc_ref[...] * (idx_ref[...] == e).astype(f32), axis=-1,
                keepdims=True)
    contrib = o * w

    @pl.when(e == 0)
    def _():
        y_ref[...] = contrib

    @pl.when(e > 0)
    def _():
        y_ref[...] = y_ref[...] + contrib


def _k4(xffn, w0, w1, w2, idx_pad, scores):
    return pl.pallas_call(
        _k4_body,
        grid=(TEN,),
        in_specs=[
            pl.BlockSpec((S, DIM), lambda e: (0, 0)),
            pl.BlockSpec((1, DIM, EDIM), lambda e: (e, 0, 0)),
            pl.BlockSpec((1, DIM, EDIM), lambda e: (e, 0, 0)),
            pl.BlockSpec((1, DIM, EDIM), lambda e: (e, 0, 0)),
            pl.BlockSpec((S, 128), lambda e: (0, 0)),
            pl.BlockSpec((S, 128), lambda e: (0, 0)),
        ],
        out_specs=pl.BlockSpec((S, DIM), lambda e: (0, 0)),
        out_shape=jax.ShapeDtypeStruct((S, DIM), f32),
    )(xffn, w0, w1, w2, idx_pad, scores)


def _k5_body(x_ref, up_ref, down_ref, y_ref, resid_ref, out_ref):
    xu = jnp.dot(x_ref[...], up_ref[...], preferred_element_type=f32)
    x1, x2 = xu[:, :SDIM], xu[:, SDIM:]
    h = (jax.nn.silu(x1) * x2).astype(bf16)
    ys = jnp.dot(h, down_ref[...], preferred_element_type=f32)
    out_ref[...] = ys + y_ref[...] + resid_ref[...]


def _k5(xffn, up_wb, down_wb, y_moe, resid):
    return pl.pallas_call(
        _k5_body,
        grid=(S // SB,),
        in_specs=[
            pl.BlockSpec((SB, DIM), lambda i: (i, 0)),
            pl.BlockSpec((DIM, 2 * SDIM), lambda i: (0, 0)),
            pl.BlockSpec((SDIM, DIM), lambda i: (0, 0)),
            pl.BlockSpec((SB, DIM), lambda i: (i, 0)),
            pl.BlockSpec((SB, DIM), lambda i: (i, 0)),
        ],
        out_specs=pl.BlockSpec((SB, DIM), lambda i: (i, 0)),
        out_shape=jax.ShapeDtypeStruct((S, DIM), f32),
    )(xffn, up_wb, down_wb, y_moe, resid)


def kernel(x_input, indices, values, attn_norm_w, qkv_w, o_w, ffn_norm_w,
           keys_w, experts_w, up_w, down_w):
    x = x_input.reshape(S, DIM)
    qkv_wb = qkv_w.astype(bf16)
    o_wb = o_w.astype(bf16)
    up_wb = up_w.astype(bf16)
    down_wb = down_w.astype(bf16)
    keys_pad = jnp.pad(keys_w, ((0, 0), (0, 128 - TEN)))
    idx_pad = jnp.pad(indices.astype(jnp.int32), ((0, 0), (0, 128 - TOPK)),
                      constant_values=TEN)
    val_pad = jnp.pad(values, ((0, 0), (0, 128 - TOPK)), constant_values=NEG)

    qkv = _k1(x, attn_norm_w.reshape(1, DIM), qkv_wb)
    attn = _k2(qkv)
    resid, xffn, scores = _k3(attn, o_wb, x, ffn_norm_w.reshape(1, DIM),
                              keys_pad, idx_pad, val_pad)
    y_moe = _k4(xffn, experts_w[0], experts_w[1], experts_w[2], idx_pad, scores)
    out = _k5(xffn, up_wb, down_wb, y_moe, resid)
    return out.reshape(B, S, DIM)


# speedup vs baseline: 1.6457x; 1.2478x over previous
import functools
import math

import jax
import jax.numpy as jnp
from jax.experimental import pallas as pl

B, S, DIM, HEADS = 1, 2048, 1024, 16
HEAD = DIM // HEADS
TEN, TOPK, EDIM, SDIM = 8, 2, 512, 1024
EPS, THETA, RSF = 1e-5, 10000.0, 1.0

SB = 256
QB = 512
NEG = -1e30

f32 = jnp.float32
bf16 = jnp.bfloat16


def _k0_body(x_ref, w_ref, xn_ref, tab_ref):
    i = pl.program_id(0)
    x = x_ref[...]
    xn = x * jax.lax.rsqrt(jnp.mean(x * x, axis=-1, keepdims=True) + EPS)
    xn_ref[...] = (xn * w_ref[...]).astype(bf16)
    half = HEAD // 2
    pos = jax.lax.broadcasted_iota(jnp.int32, (SB, half), 0).astype(f32) + i * SB
    k = jax.lax.broadcasted_iota(jnp.int32, (SB, half), 1).astype(f32)
    inv = jnp.exp(k * (2.0 / HEAD) * math.log(1.0 / THETA))
    fr = pos * inv
    tab_ref[...] = jnp.concatenate([jnp.cos(fr), jnp.sin(fr)], axis=-1)


def _k0(x, attn_norm_w):
    return pl.pallas_call(
        _k0_body,
        grid=(S // SB,),
        in_specs=[
            pl.BlockSpec((SB, DIM), lambda i: (i, 0)),
            pl.BlockSpec((1, DIM), lambda i: (0, 0)),
        ],
        out_specs=[
            pl.BlockSpec((SB, DIM), lambda i: (i, 0)),
            pl.BlockSpec((SB, HEAD), lambda i: (i, 0)),
        ],
        out_shape=[
            jax.ShapeDtypeStruct((S, DIM), bf16),
            jax.ShapeDtypeStruct((S, HEAD), f32),
        ],
    )(x, attn_norm_w)


def _prep_head(y, tab, scale):
    half = HEAD // 2
    n = jnp.sqrt(jnp.sum(y * y, axis=-1, keepdims=True))
    y = y / jnp.maximum(n, EPS)
    c, s = tab[:, :half], tab[:, half:]
    x1, x2 = y[:, :half], y[:, half:]
    return jnp.concatenate([(x1 * c + x2 * s) * scale,
                            (x2 * c - x1 * s) * scale], axis=-1)


def _k1_body(xn_ref, w_ref, tab_ref, out_ref):
    c = pl.program_id(1)
    r = jnp.dot(xn_ref[...], w_ref[...].astype(bf16),
                preferred_element_type=f32)
    tab = tab_ref[...]
    scale = 1.0 / math.sqrt(float(HEAD))

    def prepped(sc):
        return jnp.concatenate(
            [_prep_head(r[:, i * HEAD:(i + 1) * HEAD], tab, sc)
             for i in range(2)], axis=-1).astype(bf16)

    @pl.when(c < HEADS // 2)
    def _():
        out_ref[...] = prepped(scale)

    @pl.when((c >= HEADS // 2) & (c < HEADS))
    def _():
        out_ref[...] = prepped(1.0)

    @pl.when(c >= HEADS)
    def _():
        out_ref[...] = r.astype(bf16)


def _k1(xn, qkv_w, tab):
    NC = 3 * DIM // 128
    return pl.pallas_call(
        _k1_body,
        grid=(S // QB, NC),
        in_specs=[
            pl.BlockSpec((QB, DIM), lambda s, c: (s, 0)),
            pl.BlockSpec((DIM, 128), lambda s, c: (0, c)),
            pl.BlockSpec((QB, HEAD), lambda s, c: (s, 0)),
        ],
        out_specs=pl.BlockSpec((QB, 128), lambda s, c: (s, c)),
        out_shape=jax.ShapeDtypeStruct((S, 3 * DIM), bf16),
    )(xn, qkv_w, tab)


def _attn_one_head(q, k_ref, v_ref, sl, qi):

    def chunk(kj, masked):
        kc = k_ref[pl.ds(kj * QB, QB), sl]
        vc = v_ref[pl.ds(kj * QB, QB), sl]
        l = jax.lax.dot_general(q, kc, (((1,), (1,)), ((), ())),
                                preferred_element_type=f32)
        p = 1.0 + l * (1.0 + l * (0.5 + l * (1.0 / 6.0 + l * (1.0 / 24.0))))
        if masked:
            row = jax.lax.broadcasted_iota(jnp.int32, (QB, QB), 0)
            col = jax.lax.broadcasted_iota(jnp.int32, (QB, QB), 1)
            p = jnp.where(col <= row, p, 0.0)
        o = jnp.dot(p.astype(bf16), vc, preferred_element_type=f32)
        return o, jnp.sum(p, axis=-1, keepdims=True)

    def body(kj, carry):
        o_acc, s_acc = carry
        o, s = chunk(kj, masked=False)
        return o_acc + o, s_acc + s

    o_acc, s_acc = jax.lax.fori_loop(
        0, qi, body,
        (jnp.zeros((QB, HEAD), f32), jnp.zeros((QB, 1), f32)))
    o, s = chunk(qi, masked=True)
    return (o_acc + o) / (s_acc + s)


def _k2_body(q_ref, k_ref, v_ref, out_ref):
    qi = pl.program_id(1)
    outs = []
    for i in range(2):
        sl = slice(i * HEAD, (i + 1) * HEAD)
        outs.append(_attn_one_head(q_ref[:, sl], k_ref, v_ref, sl, qi))
    out_ref[...] = jnp.concatenate(outs, axis=-1).astype(bf16)


def _k2(qkv):
    HP = HEADS // 2
    return pl.pallas_call(
        _k2_body,
        grid=(HP, S // QB),
        in_specs=[
            pl.BlockSpec((QB, 2 * HEAD), lambda h, qi: (qi, h)),
            pl.BlockSpec((S, 2 * HEAD), lambda h, qi: (0, HP + h)),
            pl.BlockSpec((S, 2 * HEAD), lambda h, qi: (0, 2 * HP + h)),
        ],
        out_specs=pl.BlockSpec((QB, 2 * HEAD), lambda h, qi: (qi, h)),
        out_shape=jax.ShapeDtypeStruct((S, DIM), bf16),
    )(qkv, qkv, qkv)


def _k3_body(attn_ref, ow_ref, x_ref, fw_ref, keys_ref, idx_ref, val_ref,
             resid_ref, xffn_ref, scores_ref):
    att = jnp.dot(attn_ref[...], ow_ref[...], preferred_element_type=f32)
    resid = att + x_ref[...]
    resid_ref[...] = resid
    xn = resid * jax.lax.rsqrt(jnp.mean(resid * resid, axis=-1, keepdims=True) + EPS)
    xn = xn * fw_ref[...]
    xffn_ref[...] = xn.astype(bf16)
    tv = jnp.dot(xn, keys_ref[...], preferred_element_type=f32)
    idx = idx_ref[...]
    tvsel = jnp.zeros_like(tv)
    for e in range(TEN):
        tvsel = tvsel + tv[:, e:e + 1] * (idx == e).astype(f32)
    scores_ref[...] = jax.nn.sigmoid(val_ref[...] + tvsel) * RSF


def _k3(attn, o_wb, x_input, ffn_norm_w, keys_pad, idx_pad, val_pad):
    return pl.pallas_call(
        _k3_body,
        grid=(S // SB,),
        in_specs=[
            pl.BlockSpec((SB, DIM), lambda i: (i, 0)),
            pl.BlockSpec((DIM, DIM), lambda i: (0, 0)),
            pl.BlockSpec((SB, DIM), lambda i: (i, 0)),
            pl.BlockSpec((1, DIM), lambda i: (0, 0)),
            pl.BlockSpec((DIM, 128), lambda i: (0, 0)),
            pl.BlockSpec((SB, 128), lambda i: (i, 0)),
            pl.BlockSpec((SB, 128), lambda i: (i, 0)),
        ],
        out_specs=[
            pl.BlockSpec((SB, DIM), lambda i: (i, 0)),
            pl.BlockSpec((SB, DIM), lambda i: (i, 0)),
            pl.BlockSpec((SB, 128), lambda i: (i, 0)),
        ],
        out_shape=[
            jax.ShapeDtypeStruct((S, DIM), f32),
            jax.ShapeDtypeStruct((S, DIM), bf16),
            jax.ShapeDtypeStruct((S, 128), f32),
        ],
    )(attn, o_wb, x_input, ffn_norm_w, keys_pad, idx_pad, val_pad)


def _k4_body(x_ref, w0_ref, w1_ref, w2_ref, idx_ref, sc_ref, y_ref):
    e = pl.program_id(0)
    x = x_ref[...]
    g = jnp.dot(x, w0_ref[0].astype(bf16), preferred_element_type=f32)
    u = jnp.dot(x, w1_ref[0].astype(bf16), preferred_element_type=f32)
    h = (jax.nn.silu(g) * u).astype(bf16)
    o = jax.lax.dot_general(h, w2_ref[0].astype(bf16), (((1,), (1,)), ((), ())),
                            preferred_element_type=f32)
    w = jnp.sum(sc_ref[...] * (idx_ref[...] == e).astype(f32), axis=-1,
                keepdims=True)
    contrib = o * w

    @pl.when(e == 0)
    def _():
        y_ref[...] = contrib

    @pl.when(e > 0)
    def _():
        y_ref[...] = y_ref[...] + contrib


def _k4(xffn, w0, w1, w2, idx_pad, scores):
    return pl.pallas_call(
        _k4_body,
        grid=(TEN,),
        in_specs=[
            pl.BlockSpec((S, DIM), lambda e: (0, 0)),
            pl.BlockSpec((1, DIM, EDIM), lambda e: (e, 0, 0)),
            pl.BlockSpec((1, DIM, EDIM), lambda e: (e, 0, 0)),
            pl.BlockSpec((1, DIM, EDIM), lambda e: (e, 0, 0)),
            pl.BlockSpec((S, 128), lambda e: (0, 0)),
            pl.BlockSpec((S, 128), lambda e: (0, 0)),
        ],
        out_specs=pl.BlockSpec((S, DIM), lambda e: (0, 0)),
        out_shape=jax.ShapeDtypeStruct((S, DIM), f32),
    )(xffn, w0, w1, w2, idx_pad, scores)


def _k5_body(x_ref, up_ref, down_ref, y_ref, resid_ref, out_ref):
    xu = jnp.dot(x_ref[...], up_ref[...], preferred_element_type=f32)
    x1, x2 = xu[:, :SDIM], xu[:, SDIM:]
    h = (jax.nn.silu(x1) * x2).astype(bf16)
    ys = jnp.dot(h, down_ref[...], preferred_element_type=f32)
    out_ref[...] = ys + y_ref[...] + resid_ref[...]


def _k5(xffn, up_wb, down_wb, y_moe, resid):
    return pl.pallas_call(
        _k5_body,
        grid=(S // SB,),
        in_specs=[
            pl.BlockSpec((SB, DIM), lambda i: (i, 0)),
            pl.BlockSpec((DIM, 2 * SDIM), lambda i: (0, 0)),
            pl.BlockSpec((SDIM, DIM), lambda i: (0, 0)),
            pl.BlockSpec((SB, DIM), lambda i: (i, 0)),
            pl.BlockSpec((SB, DIM), lambda i: (i, 0)),
        ],
        out_specs=pl.BlockSpec((SB, DIM), lambda i: (i, 0)),
        out_shape=jax.ShapeDtypeStruct((S, DIM), f32),
    )(xffn, up_wb, down_wb, y_moe, resid)


def kernel(x_input, indices, values, attn_norm_w, qkv_w, o_w, ffn_norm_w,
           keys_w, experts_w, up_w, down_w):
    x = x_input.reshape(S, DIM)
    o_wb = o_w.astype(bf16)
    up_wb = up_w.astype(bf16)
    down_wb = down_w.astype(bf16)
    keys_pad = jnp.pad(keys_w, ((0, 0), (0, 128 - TEN)))
    idx_pad = jnp.pad(indices.astype(jnp.int32), ((0, 0), (0, 128 - TOPK)),
                      constant_values=TEN)
    val_pad = jnp.pad(values, ((0, 0), (0, 128 - TOPK)), constant_values=NEG)

    xn, tab = _k0(x, attn_norm_w.reshape(1, DIM))
    qkv = _k1(xn, qkv_w, tab)
    attn = _k2(qkv)
    resid, xffn, scores = _k3(attn, o_wb, x, ffn_norm_w.reshape(1, DIM),
                              keys_pad, idx_pad, val_pad)
    y_moe = _k4(xffn, experts_w[0], experts_w[1], experts_w[2], idx_pad, scores)
    out = _k5(xffn, up_wb, down_wb, y_moe, resid)
    return out.reshape(B, S, DIM)


# speedup vs baseline: 1.9568x; 1.1891x over previous
import functools
import math

import jax
import jax.numpy as jnp
from jax.experimental import pallas as pl

B, S, DIM, HEADS = 1, 2048, 1024, 16
HEAD = DIM // HEADS
TEN, TOPK, EDIM, SDIM = 8, 2, 512, 1024
EPS, THETA, RSF = 1e-5, 10000.0, 1.0

SB = 256
QB = 512
NEG = -1e30

f32 = jnp.float32
bf16 = jnp.bfloat16


KC = 256
HALF = HEAD // 2


def _k0_body(x_ref, w_ref, xn_ref, ta_ref, tb_ref):
    i = pl.program_id(0)
    x = x_ref[...]
    xn = x * jax.lax.rsqrt(jnp.mean(x * x, axis=-1, keepdims=True) + EPS)
    xn_ref[...] = (xn * w_ref[...]).astype(bf16)
    pos = jax.lax.broadcasted_iota(jnp.int32, (SB, KC), 0).astype(f32) + i * SB
    colv = jax.lax.broadcasted_iota(jnp.int32, (SB, KC), 1)
    k = (colv % HALF).astype(f32)
    inv = jnp.exp(k * (2.0 / HEAD) * math.log(1.0 / THETA))
    fr = pos * inv
    ta_ref[...] = jnp.cos(fr)
    sgn = jnp.where((colv // HALF) % 2 == 0, 1.0, -1.0)
    tb_ref[...] = jnp.sin(fr) * sgn


def _k0(x, attn_norm_w):
    return pl.pallas_call(
        _k0_body,
        grid=(S // SB,),
        in_specs=[
            pl.BlockSpec((SB, DIM), lambda i: (i, 0)),
            pl.BlockSpec((1, DIM), lambda i: (0, 0)),
        ],
        out_specs=[
            pl.BlockSpec((SB, DIM), lambda i: (i, 0)),
            pl.BlockSpec((SB, KC), lambda i: (i, 0)),
            pl.BlockSpec((SB, KC), lambda i: (i, 0)),
        ],
        out_shape=[
            jax.ShapeDtypeStruct((S, DIM), bf16),
            jax.ShapeDtypeStruct((S, KC), f32),
            jax.ShapeDtypeStruct((S, KC), f32),
        ],
    )(x, attn_norm_w)


NH_BLK = KC // HEAD


def _k1_prep(r, ta, tb, scale):
    col = jax.lax.broadcasted_iota(jnp.int32, (KC, KC), 0)
    row = jax.lax.broadcasted_iota(jnp.int32, (KC, KC), 1)
    gmat = (col // HEAD == row // HEAD).astype(f32)
    pmat = (row == (col // HEAD) * HEAD + (col % HEAD + HALF) % HEAD).astype(f32)
    z = r * r
    ss = jax.lax.dot_general(z, gmat, (((1,), (0,)), ((), ())),
                             preferred_element_type=f32)
    yn = r / jnp.maximum(jnp.sqrt(ss), EPS)
    sw = jax.lax.dot_general(yn, pmat, (((1,), (0,)), ((), ())),
                             preferred_element_type=f32)
    return ((yn * ta + sw * tb) * scale).astype(bf16)


def _k1_body(xn_ref, w_ref, ta_ref, tb_ref, out_ref):
    c = pl.program_id(1)
    r = jnp.dot(xn_ref[...], w_ref[...].astype(bf16),
                preferred_element_type=f32)
    nq = DIM // KC
    scale = 1.0 / math.sqrt(float(HEAD))

    @pl.when(c < nq)
    def _():
        out_ref[...] = _k1_prep(r, ta_ref[...], tb_ref[...], scale)

    @pl.when((c >= nq) & (c < 2 * nq))
    def _():
        out_ref[...] = _k1_prep(r, ta_ref[...], tb_ref[...], 1.0)

    @pl.when(c >= 2 * nq)
    def _():
        out_ref[...] = r.astype(bf16)


def _k1(xn, qkv_w, ta, tb):
    NC = 3 * DIM // KC
    return pl.pallas_call(
        _k1_body,
        grid=(S // QB, NC),
        in_specs=[
            pl.BlockSpec((QB, DIM), lambda s, c: (s, 0)),
            pl.BlockSpec((DIM, KC), lambda s, c: (0, c)),
            pl.BlockSpec((QB, KC), lambda s, c: (s, 0)),
            pl.BlockSpec((QB, KC), lambda s, c: (s, 0)),
        ],
        out_specs=pl.BlockSpec((QB, KC), lambda s, c: (s, c)),
        out_shape=jax.ShapeDtypeStruct((S, 3 * DIM), bf16),
    )(xn, qkv_w, ta, tb)


def _attn_one_head(q, k_ref, v_ref, sl, qi):

    def chunk(kj, masked):
        kc = k_ref[pl.ds(kj * QB, QB), sl]
        vc = v_ref[pl.ds(kj * QB, QB), sl]
        l = jax.lax.dot_general(q, kc, (((1,), (1,)), ((), ())),
                                preferred_element_type=f32)
        p = 1.0 + l * (1.0 + l * (0.5 + l * (1.0 / 6.0 + l * (1.0 / 24.0))))
        if masked:
            row = jax.lax.broadcasted_iota(jnp.int32, (QB, QB), 0)
            col = jax.lax.broadcasted_iota(jnp.int32, (QB, QB), 1)
            p = jnp.where(col <= row, p, 0.0)
        o = jnp.dot(p.astype(bf16), vc, preferred_element_type=f32)
        return o, jnp.sum(p, axis=-1, keepdims=True)

    def body(kj, carry):
        o_acc, s_acc = carry
        o, s = chunk(kj, masked=False)
        return o_acc + o, s_acc + s

    o_acc, s_acc = jax.lax.fori_loop(
        0, qi, body,
        (jnp.zeros((QB, HEAD), f32), jnp.zeros((QB, 1), f32)))
    o, s = chunk(qi, masked=True)
    return (o_acc + o) / (s_acc + s)


def _k2_body(q_ref, k_ref, v_ref, out_ref):
    qi = pl.program_id(1)
    outs = []
    for i in range(2):
        sl = slice(i * HEAD, (i + 1) * HEAD)
        outs.append(_attn_one_head(q_ref[:, sl], k_ref, v_ref, sl, qi))
    out_ref[...] = jnp.concatenate(outs, axis=-1).astype(bf16)


def _k2(qkv):
    HP = HEADS // 2
    return pl.pallas_call(
        _k2_body,
        grid=(HP, S // QB),
        in_specs=[
            pl.BlockSpec((QB, 2 * HEAD), lambda h, qi: (qi, h)),
            pl.BlockSpec((S, 2 * HEAD), lambda h, qi: (0, HP + h)),
            pl.BlockSpec((S, 2 * HEAD), lambda h, qi: (0, 2 * HP + h)),
        ],
        out_specs=pl.BlockSpec((QB, 2 * HEAD), lambda h, qi: (qi, h)),
        out_shape=jax.ShapeDtypeStruct((S, DIM), bf16),
    )(qkv, qkv, qkv)


def _k3_body(attn_ref, ow_ref, x_ref, fw_ref, keys_ref, idx_ref, val_ref,
             resid_ref, xffn_ref, scores_ref):
    att = jnp.dot(attn_ref[...], ow_ref[...], preferred_element_type=f32)
    resid = att + x_ref[...]
    resid_ref[...] = resid
    xn = resid * jax.lax.rsqrt(jnp.mean(resid * resid, axis=-1, keepdims=True) + EPS)
    xn = xn * fw_ref[...]
    xffn_ref[...] = xn.astype(bf16)
    tv = jnp.dot(xn, keys_ref[...], preferred_element_type=f32)
    idx = idx_ref[...]
    tvsel = jnp.zeros_like(tv)
    for e in range(TEN):
        tvsel = tvsel + tv[:, e:e + 1] * (idx == e).astype(f32)
    scores_ref[...] = jax.nn.sigmoid(val_ref[...] + tvsel) * RSF


def _k3(attn, o_wb, x_input, ffn_norm_w, keys_pad, idx_pad, val_pad):
    return pl.pallas_call(
        _k3_body,
        grid=(S // SB,),
        in_specs=[
            pl.BlockSpec((SB, DIM), lambda i: (i, 0)),
            pl.BlockSpec((DIM, DIM), lambda i: (0, 0)),
            pl.BlockSpec((SB, DIM), lambda i: (i, 0)),
            pl.BlockSpec((1, DIM), lambda i: (0, 0)),
            pl.BlockSpec((DIM, 128), lambda i: (0, 0)),
            pl.BlockSpec((SB, 128), lambda i: (i, 0)),
            pl.BlockSpec((SB, 128), lambda i: (i, 0)),
        ],
        out_specs=[
            pl.BlockSpec((SB, DIM), lambda i: (i, 0)),
            pl.BlockSpec((SB, DIM), lambda i: (i, 0)),
            pl.BlockSpec((SB, 128), lambda i: (i, 0)),
        ],
        out_shape=[
            jax.ShapeDtypeStruct((S, DIM), f32),
            jax.ShapeDtypeStruct((S, DIM), bf16),
            jax.ShapeDtypeStruct((S, 128), f32),
        ],
    )(attn, o_wb, x_input, ffn_norm_w, keys_pad, idx_pad, val_pad)


def _k4_body(x_ref, w0_ref, w1_ref, w2_ref, idx_ref, sc_ref, y_ref):
    e = pl.program_id(0)
    x = x_ref[...]
    g = jnp.dot(x, w0_ref[0].astype(bf16), preferred_element_type=f32)
    u = jnp.dot(x, w1_ref[0].astype(bf16), preferred_element_type=f32)
    h = (jax.nn.silu(g) * u).astype(bf16)
    o = jax.lax.dot_general(h, w2_ref[0].astype(bf16), (((1,), (1,)), ((), ())),
                            preferred_element_type=f32)
    w = jnp.sum(sc_ref[...] * (idx_ref[...] == e).astype(f32), axis=-1,
                keepdims=True)
    contrib = o * w

    @pl.when(e == 0)
    def _():
        y_ref[...] = contrib

    @pl.when(e > 0)
    def _():
        y_ref[...] = y_ref[...] + contrib


def _k4(xffn, w0, w1, w2, idx_pad, scores):
    return pl.pallas_call(
        _k4_body,
        grid=(TEN,),
        in_specs=[
            pl.BlockSpec((S, DIM), lambda e: (0, 0)),
            pl.BlockSpec((1, DIM, EDIM), lambda e: (e, 0, 0)),
            pl.BlockSpec((1, DIM, EDIM), lambda e: (e, 0, 0)),
            pl.BlockSpec((1, DIM, EDIM), lambda e: (e, 0, 0)),
            pl.BlockSpec((S, 128), lambda e: (0, 0)),
            pl.BlockSpec((S, 128), lambda e: (0, 0)),
        ],
        out_specs=pl.BlockSpec((S, DIM), lambda e: (0, 0)),
        out_shape=jax.ShapeDtypeStruct((S, DIM), f32),
    )(xffn, w0, w1, w2, idx_pad, scores)


def _k5_body(x_ref, up_ref, down_ref, y_ref, resid_ref, out_ref):
    xu = jnp.dot(x_ref[...], up_ref[...], preferred_element_type=f32)
    x1, x2 = xu[:, :SDIM], xu[:, SDIM:]
    h = (jax.nn.silu(x1) * x2).astype(bf16)
    ys = jnp.dot(h, down_ref[...], preferred_element_type=f32)
    out_ref[...] = ys + y_ref[...] + resid_ref[...]


def _k5(xffn, up_wb, down_wb, y_moe, resid):
    return pl.pallas_call(
        _k5_body,
        grid=(S // SB,),
        in_specs=[
            pl.BlockSpec((SB, DIM), lambda i: (i, 0)),
            pl.BlockSpec((DIM, 2 * SDIM), lambda i: (0, 0)),
            pl.BlockSpec((SDIM, DIM), lambda i: (0, 0)),
            pl.BlockSpec((SB, DIM), lambda i: (i, 0)),
            pl.BlockSpec((SB, DIM), lambda i: (i, 0)),
        ],
        out_specs=pl.BlockSpec((SB, DIM), lambda i: (i, 0)),
        out_shape=jax.ShapeDtypeStruct((S, DIM), f32),
    )(xffn, up_wb, down_wb, y_moe, resid)


def kernel(x_input, indices, values, attn_norm_w, qkv_w, o_w, ffn_norm_w,
           keys_w, experts_w, up_w, down_w):
    x = x_input.reshape(S, DIM)
    o_wb = o_w.astype(bf16)
    up_wb = up_w.astype(bf16)
    down_wb = down_w.astype(bf16)
    keys_pad = jnp.pad(keys_w, ((0, 0), (0, 128 - TEN)))
    idx_pad = jnp.pad(indices.astype(jnp.int32), ((0, 0), (0, 128 - TOPK)),
                      constant_values=TEN)
    val_pad = jnp.pad(values, ((0, 0), (0, 128 - TOPK)), constant_values=NEG)

    xn, ta, tb = _k0(x, attn_norm_w.reshape(1, DIM))
    qkv = _k1(xn, qkv_w, ta, tb)
    attn = _k2(qkv)
    resid, xffn, scores = _k3(attn, o_wb, x, ffn_norm_w.reshape(1, DIM),
                              keys_pad, idx_pad, val_pad)
    y_moe = _k4(xffn, experts_w[0], experts_w[1], experts_w[2], idx_pad, scores)
    out = _k5(xffn, up_wb, down_wb, y_moe, resid)
    return out.reshape(B, S, DIM)


# speedup vs baseline: 2.0471x; 1.0461x over previous
import functools
import math

import jax
import jax.numpy as jnp
from jax.experimental import pallas as pl

B, S, DIM, HEADS = 1, 2048, 1024, 16
HEAD = DIM // HEADS
TEN, TOPK, EDIM, SDIM = 8, 2, 512, 1024
EPS, THETA, RSF = 1e-5, 10000.0, 1.0

SB = 256
QB = 512
NEG = -1e30

f32 = jnp.float32
bf16 = jnp.bfloat16


KC = 256
HALF = HEAD // 2


def _k0_body(x_ref, w_ref, xn_ref, ta_ref, tb_ref):
    i = pl.program_id(0)
    x = x_ref[...]
    xn = x * jax.lax.rsqrt(jnp.mean(x * x, axis=-1, keepdims=True) + EPS)
    xn_ref[...] = (xn * w_ref[...]).astype(bf16)
    pos = jax.lax.broadcasted_iota(jnp.int32, (SB, KC), 0).astype(f32) + i * SB
    colv = jax.lax.broadcasted_iota(jnp.int32, (SB, KC), 1)
    k = (colv % HALF).astype(f32)
    inv = jnp.exp(k * (2.0 / HEAD) * math.log(1.0 / THETA))
    fr = pos * inv
    ta_ref[...] = jnp.cos(fr)
    sgn = jnp.where((colv // HALF) % 2 == 0, 1.0, -1.0)
    tb_ref[...] = jnp.sin(fr) * sgn


def _k0(x, attn_norm_w):
    return pl.pallas_call(
        _k0_body,
        grid=(S // SB,),
        in_specs=[
            pl.BlockSpec((SB, DIM), lambda i: (i, 0)),
            pl.BlockSpec((1, DIM), lambda i: (0, 0)),
        ],
        out_specs=[
            pl.BlockSpec((SB, DIM), lambda i: (i, 0)),
            pl.BlockSpec((SB, KC), lambda i: (i, 0)),
            pl.BlockSpec((SB, KC), lambda i: (i, 0)),
        ],
        out_shape=[
            jax.ShapeDtypeStruct((S, DIM), bf16),
            jax.ShapeDtypeStruct((S, KC), f32),
            jax.ShapeDtypeStruct((S, KC), f32),
        ],
    )(x, attn_norm_w)


NH_BLK = KC // HEAD


def _k1_prep(r, ta, tb, scale):
    col = jax.lax.broadcasted_iota(jnp.int32, (KC, KC), 0)
    row = jax.lax.broadcasted_iota(jnp.int32, (KC, KC), 1)
    gmat = (col // HEAD == row // HEAD).astype(f32)
    pmat = (row == (col // HEAD) * HEAD + (col % HEAD + HALF) % HEAD).astype(f32)
    z = r * r
    ss = jax.lax.dot_general(z, gmat, (((1,), (0,)), ((), ())),
                             preferred_element_type=f32)
    yn = r / jnp.maximum(jnp.sqrt(ss), EPS)
    sw = jax.lax.dot_general(yn, pmat, (((1,), (0,)), ((), ())),
                             preferred_element_type=f32)
    return ((yn * ta + sw * tb) * scale).astype(bf16)


def _k1_body(xn_ref, w_ref, ta_ref, tb_ref, out_ref):
    c = pl.program_id(1)
    r = jnp.dot(xn_ref[...], w_ref[...].astype(bf16),
                preferred_element_type=f32)
    nq = DIM // KC
    scale = 1.0 / math.sqrt(float(HEAD))

    @pl.when(c < nq)
    def _():
        out_ref[...] = _k1_prep(r, ta_ref[...], tb_ref[...], scale)

    @pl.when((c >= nq) & (c < 2 * nq))
    def _():
        out_ref[...] = _k1_prep(r, ta_ref[...], tb_ref[...], 1.0)

    @pl.when(c >= 2 * nq)
    def _():
        out_ref[...] = r.astype(bf16)


def _k1(xn, qkv_w, ta, tb):
    NC = 3 * DIM // KC
    return pl.pallas_call(
        _k1_body,
        grid=(S // QB, NC),
        in_specs=[
            pl.BlockSpec((QB, DIM), lambda s, c: (s, 0)),
            pl.BlockSpec((DIM, KC), lambda s, c: (0, c)),
            pl.BlockSpec((QB, KC), lambda s, c: (s, 0)),
            pl.BlockSpec((QB, KC), lambda s, c: (s, 0)),
        ],
        out_specs=pl.BlockSpec((QB, KC), lambda s, c: (s, c)),
        out_shape=jax.ShapeDtypeStruct((S, 3 * DIM), bf16),
    )(xn, qkv_w, ta, tb)


def _attn_one_head(q, k_ref, v_ref, sl, qi):

    def chunk(kj, masked):
        kc = k_ref[pl.ds(kj * QB, QB), sl]
        vc = v_ref[pl.ds(kj * QB, QB), sl]
        l = jax.lax.dot_general(q, kc, (((1,), (1,)), ((), ())),
                                preferred_element_type=f32).astype(bf16)
        one = jnp.array(1.0, bf16)
        p = one + l * (one + l * (jnp.array(0.5, bf16)
                                  + l * jnp.array(1.0 / 6.0, bf16)))
        if masked:
            row = jax.lax.broadcasted_iota(jnp.int32, (QB, QB), 0)
            col = jax.lax.broadcasted_iota(jnp.int32, (QB, QB), 1)
            p = jnp.where(col <= row, p, jnp.array(0.0, bf16))
        o = jnp.dot(p, vc, preferred_element_type=f32)
        return o, jnp.sum(p, axis=-1, keepdims=True).astype(f32)

    def body(kj, carry):
        o_acc, s_acc = carry
        o, s = chunk(kj, masked=False)
        return o_acc + o, s_acc + s

    o_acc, s_acc = jax.lax.fori_loop(
        0, qi, body,
        (jnp.zeros((QB, HEAD), f32), jnp.zeros((QB, 1), f32)))
    o, s = chunk(qi, masked=True)
    return (o_acc + o) / (s_acc + s)


def _k2_body(q_ref, k_ref, v_ref, out_ref):
    qi = pl.program_id(1)
    outs = []
    for i in range(2):
        sl = slice(i * HEAD, (i + 1) * HEAD)
        outs.append(_attn_one_head(q_ref[:, sl], k_ref, v_ref, sl, qi))
    out_ref[...] = jnp.concatenate(outs, axis=-1).astype(bf16)


def _k2(qkv):
    HP = HEADS // 2
    return pl.pallas_call(
        _k2_body,
        grid=(HP, S // QB),
        in_specs=[
            pl.BlockSpec((QB, 2 * HEAD), lambda h, qi: (qi, h)),
            pl.BlockSpec((S, 2 * HEAD), lambda h, qi: (0, HP + h)),
            pl.BlockSpec((S, 2 * HEAD), lambda h, qi: (0, 2 * HP + h)),
        ],
        out_specs=pl.BlockSpec((QB, 2 * HEAD), lambda h, qi: (qi, h)),
        out_shape=jax.ShapeDtypeStruct((S, DIM), bf16),
    )(qkv, qkv, qkv)


def _k3_body(attn_ref, ow_ref, x_ref, fw_ref, keys_ref, idx_ref, val_ref,
             resid_ref, xffn_ref, scores_ref):
    att = jnp.dot(attn_ref[...], ow_ref[...], preferred_element_type=f32)
    resid = att + x_ref[...]
    resid_ref[...] = resid
    xn = resid * jax.lax.rsqrt(jnp.mean(resid * resid, axis=-1, keepdims=True) + EPS)
    xn = xn * fw_ref[...]
    xffn_ref[...] = xn.astype(bf16)
    tv = jnp.dot(xn, keys_ref[...], preferred_element_type=f32)
    idx = idx_ref[...]
    tvsel = jnp.zeros_like(tv)
    for e in range(TEN):
        tvsel = tvsel + tv[:, e:e + 1] * (idx == e).astype(f32)
    scores_ref[...] = jax.nn.sigmoid(val_ref[...] + tvsel) * RSF


def _k3(attn, o_wb, x_input, ffn_norm_w, keys_pad, idx_pad, val_pad):
    return pl.pallas_call(
        _k3_body,
        grid=(S // SB,),
        in_specs=[
            pl.BlockSpec((SB, DIM), lambda i: (i, 0)),
            pl.BlockSpec((DIM, DIM), lambda i: (0, 0)),
            pl.BlockSpec((SB, DIM), lambda i: (i, 0)),
            pl.BlockSpec((1, DIM), lambda i: (0, 0)),
            pl.BlockSpec((DIM, 128), lambda i: (0, 0)),
            pl.BlockSpec((SB, 128), lambda i: (i, 0)),
            pl.BlockSpec((SB, 128), lambda i: (i, 0)),
        ],
        out_specs=[
            pl.BlockSpec((SB, DIM), lambda i: (i, 0)),
            pl.BlockSpec((SB, DIM), lambda i: (i, 0)),
            pl.BlockSpec((SB, 128), lambda i: (i, 0)),
        ],
        out_shape=[
            jax.ShapeDtypeStruct((S, DIM), f32),
            jax.ShapeDtypeStruct((S, DIM), bf16),
            jax.ShapeDtypeStruct((S, 128), f32),
        ],
    )(attn, o_wb, x_input, ffn_norm_w, keys_pad, idx_pad, val_pad)


def _k4_body(x_ref, w0_ref, w1_ref, w2_ref, idx_ref, sc_ref, y_ref):
    e = pl.program_id(0)
    x = x_ref[...]
    g = jnp.dot(x, w0_ref[0].astype(bf16), preferred_element_type=f32)
    u = jnp.dot(x, w1_ref[0].astype(bf16), preferred_element_type=f32)
    h = (jax.nn.silu(g) * u).astype(bf16)
    o = jax.lax.dot_general(h, w2_ref[0].astype(bf16), (((1,), (1,)), ((), ())),
                            preferred_element_type=f32)
    w = jnp.sum(sc_ref[...] * (idx_ref[...] == e).astype(f32), axis=-1,
                keepdims=True)
    contrib = o * w

    @pl.when(e == 0)
    def _():
        y_ref[...] = contrib

    @pl.when(e > 0)
    def _():
        y_ref[...] = y_ref[...] + contrib


def _k4(xffn, w0, w1, w2, idx_pad, scores):
    return pl.pallas_call(
        _k4_body,
        grid=(TEN,),
        in_specs=[
            pl.BlockSpec((S, DIM), lambda e: (0, 0)),
            pl.BlockSpec((1, DIM, EDIM), lambda e: (e, 0, 0)),
            pl.BlockSpec((1, DIM, EDIM), lambda e: (e, 0, 0)),
            pl.BlockSpec((1, DIM, EDIM), lambda e: (e, 0, 0)),
            pl.BlockSpec((S, 128), lambda e: (0, 0)),
            pl.BlockSpec((S, 128), lambda e: (0, 0)),
        ],
        out_specs=pl.BlockSpec((S, DIM), lambda e: (0, 0)),
        out_shape=jax.ShapeDtypeStruct((S, DIM), f32),
    )(xffn, w0, w1, w2, idx_pad, scores)


def _k5_body(x_ref, up_ref, down_ref, y_ref, resid_ref, out_ref):
    xu = jnp.dot(x_ref[...], up_ref[...], preferred_element_type=f32)
    x1, x2 = xu[:, :SDIM], xu[:, SDIM:]
    h = (jax.nn.silu(x1) * x2).astype(bf16)
    ys = jnp.dot(h, down_ref[...], preferred_element_type=f32)
    out_ref[...] = ys + y_ref[...] + resid_ref[...]


def _k5(xffn, up_wb, down_wb, y_moe, resid):
    return pl.pallas_call(
        _k5_body,
        grid=(S // SB,),
        in_specs=[
            pl.BlockSpec((SB, DIM), lambda i: (i, 0)),
            pl.BlockSpec((DIM, 2 * SDIM), lambda i: (0, 0)),
            pl.BlockSpec((SDIM, DIM), lambda i: (0, 0)),
            pl.BlockSpec((SB, DIM), lambda i: (i, 0)),
            pl.BlockSpec((SB, DIM), lambda i: (i, 0)),
        ],
        out_specs=pl.BlockSpec((SB, DIM), lambda i: (i, 0)),
        out_shape=jax.ShapeDtypeStruct((S, DIM), f32),
    )(xffn, up_wb, down_wb, y_moe, resid)


def kernel(x_input, indices, values, attn_norm_w, qkv_w, o_w, ffn_norm_w,
           keys_w, experts_w, up_w, down_w):
    x = x_input.reshape(S, DIM)
    o_wb = o_w.astype(bf16)
    up_wb = up_w.astype(bf16)
    down_wb = down_w.astype(bf16)
    keys_pad = jnp.pad(keys_w, ((0, 0), (0, 128 - TEN)))
    idx_pad = jnp.pad(indices.astype(jnp.int32), ((0, 0), (0, 128 - TOPK)),
                      constant_values=TEN)
    val_pad = jnp.pad(values, ((0, 0), (0, 128 - TOPK)), constant_values=NEG)

    xn, ta, tb = _k0(x, attn_norm_w.reshape(1, DIM))
    qkv = _k1(xn, qkv_w, ta, tb)
    attn = _k2(qkv)
    resid, xffn, scores = _k3(attn, o_wb, x, ffn_norm_w.reshape(1, DIM),
                              keys_pad, idx_pad, val_pad)
    y_moe = _k4(xffn, experts_w[0], experts_w[1], experts_w[2], idx_pad, scores)
    out = _k5(xffn, up_wb, down_wb, y_moe, resid)
    return out.reshape(B, S, DIM)
